# Initial kernel scaffold; baseline (speedup 1.0000x reference)
import jax
import jax.numpy as jnp
from jax.experimental import pallas as pl


def kernel(x):
    raise NotImplementedError("write your pallas kernel here")


# speedup vs baseline: 17.1779x; 17.1779x over previous
import jax
import jax.numpy as jnp
from jax.experimental import pallas as pl
from jax.experimental.pallas import tpu as pltpu

_LIFETIME_K = 5
_NEG = -3.0e38


def _stage_a(x_ref, win_ref, i1_ref, i2_ref):
    y = x_ref[0]
    c, w, h = y.shape
    m = jnp.max(jnp.max(y, axis=2), axis=1)
    eq = y == m[:, None, None]
    fi = (jax.lax.broadcasted_iota(jnp.int32, (c, w, h), 1) * h
          + jax.lax.broadcasted_iota(jnp.int32, (c, w, h), 2))
    i1 = jnp.min(jnp.min(jnp.where(eq, fi, w * h), axis=2), axis=1)
    i2 = jnp.max(jnp.max(jnp.where(eq, fi, -1), axis=2), axis=1)
    win_ref[0, 0, :] = m
    i1_ref[0, 0, :] = i1
    i2_ref[0, 0, :] = i2


def _stage_b(win_full_ref, win_ref, i1_ref, i2_ref, out_ref, m2_ref):
    b = pl.program_id(0)

    @pl.when(b == 0)
    def _():
        w = win_full_ref[:, 0, :]
        ge = (w[:, None, :] >= w[None, :, :]).astype(jnp.float32)
        cnt = jnp.sum(ge, axis=0)
        thr = jnp.max(jnp.where(cnt >= _LIFETIME_K, w, _NEG), axis=0)
        m2_ref[...] = (w >= thr[None, :]).astype(jnp.float32)

    _, c, w, h = out_ref.shape
    wrow = win_ref[0, 0, :]
    i1 = i1_ref[0, 0, :]
    i2 = i2_ref[0, 0, :]
    m2row = m2_ref[pl.ds(b, 1), :][0]
    fi = (jax.lax.broadcasted_iota(jnp.int32, (c, w, h), 1) * h
          + jax.lax.broadcasted_iota(jnp.int32, (c, w, h), 2))
    keep = (fi == i1[:, None, None]) | (fi == i2[:, None, None])
    val = (wrow * m2row)[:, None, None]
    out_ref[0] = jnp.where(keep, val, 0.0)


def kernel(x):
    b, c, w, h = x.shape
    f32 = jnp.float32
    winners, i1, i2 = pl.pallas_call(
        _stage_a,
        grid=(b,),
        in_specs=[pl.BlockSpec((1, c, w, h), lambda i: (i, 0, 0, 0))],
        out_specs=[pl.BlockSpec((1, 1, c), lambda i: (i, 0, 0))] * 3,
        out_shape=[jax.ShapeDtypeStruct((b, 1, c), f32),
                   jax.ShapeDtypeStruct((b, 1, c), jnp.int32),
                   jax.ShapeDtypeStruct((b, 1, c), jnp.int32)],
    )(x)
    out = pl.pallas_call(
        _stage_b,
        grid=(b,),
        in_specs=[pl.BlockSpec((b, 1, c), lambda i: (0, 0, 0)),
                  pl.BlockSpec((1, 1, c), lambda i: (i, 0, 0)),
                  pl.BlockSpec((1, 1, c), lambda i: (i, 0, 0)),
                  pl.BlockSpec((1, 1, c), lambda i: (i, 0, 0))],
        out_specs=pl.BlockSpec((1, c, w, h), lambda i: (i, 0, 0, 0)),
        out_shape=jax.ShapeDtypeStruct((b, c, w, h), f32),
        scratch_shapes=[pltpu.VMEM((b, c), f32)],
    )(winners, winners, i1, i2)
    return out


# speedup vs baseline: 21.5083x; 1.2521x over previous
import jax
import jax.numpy as jnp
from jax.experimental import pallas as pl
from jax.experimental.pallas import tpu as pltpu

_LIFETIME_K = 5
_NEG = -3.0e38


def _make_iotas(w, h):
    fi_i = (jax.lax.broadcasted_iota(jnp.int32, (1, w, h), 1) * h
            + jax.lax.broadcasted_iota(jnp.int32, (1, w, h), 2))
    fi = fi_i.astype(jnp.float32)
    return fi, (w * h - 1.0) - fi


def _stage_a(x_ref, win_ref, i1_ref, i2_ref, fi_ref, rfi_ref):
    bb, c, w, h = x_ref.shape

    @pl.when(pl.program_id(0) == 0)
    def _():
        fi, rfi = _make_iotas(w, h)
        fi_ref[...] = fi
        rfi_ref[...] = rfi

    fi = fi_ref[...]
    rfi = rfi_ref[...]
    for j in range(bb):
        y = x_ref[j]
        m = jnp.max(jnp.max(y, axis=2), axis=1)
        eqf = (y == m[:, None, None]).astype(jnp.float32)
        i2 = jnp.max(jnp.max(eqf * fi, axis=2), axis=1)
        i1 = (w * h - 1.0) - jnp.max(jnp.max(eqf * rfi, axis=2), axis=1)
        win_ref[j, 0, :] = m
        i1_ref[j, 0, :] = i1
        i2_ref[j, 0, :] = i2


def _stage_b(win_full_ref, win_ref, i1_ref, i2_ref, out_ref, m2_ref, fi_ref):
    bb = out_ref.shape[0]
    step = pl.program_id(0)

    @pl.when(step == 0)
    def _():
        w = win_full_ref[:, 0, :]
        ge = (w[:, None, :] >= w[None, :, :]).astype(jnp.float32)
        cnt = jnp.sum(ge, axis=0)
        thr = jnp.max(jnp.where(cnt >= _LIFETIME_K, w, _NEG), axis=0)
        m2_ref[...] = (w >= thr[None, :]).astype(jnp.float32)
        fi, _ = _make_iotas(out_ref.shape[2], out_ref.shape[3])
        fi_ref[...] = fi

    fi = fi_ref[...]
    for j in range(bb):
        b = step * bb + j
        wrow = win_ref[j, 0, :]
        i1 = i1_ref[j, 0, :]
        i2 = i2_ref[j, 0, :]
        m2row = m2_ref[pl.ds(b, 1), :][0]
        keep = (fi == i1[:, None, None]) | (fi == i2[:, None, None])
        val = (wrow * m2row)[:, None, None]
        out_ref[j] = jnp.where(keep, val, 0.0)


def kernel(x):
    b, c, w, h = x.shape
    f32 = jnp.float32
    bb = 2
    grid = b // bb
    winners, i1, i2 = pl.pallas_call(
        _stage_a,
        grid=(grid,),
        in_specs=[pl.BlockSpec((bb, c, w, h), lambda i: (i, 0, 0, 0))],
        out_specs=[pl.BlockSpec((bb, 1, c), lambda i: (i, 0, 0))] * 3,
        out_shape=[jax.ShapeDtypeStruct((b, 1, c), f32)] * 3,
        scratch_shapes=[pltpu.VMEM((1, w, h), f32)] * 2,
    )(x)
    out = pl.pallas_call(
        _stage_b,
        grid=(grid,),
        in_specs=[pl.BlockSpec((b, 1, c), lambda i: (0, 0, 0)),
                  pl.BlockSpec((bb, 1, c), lambda i: (i, 0, 0)),
                  pl.BlockSpec((bb, 1, c), lambda i: (i, 0, 0)),
                  pl.BlockSpec((bb, 1, c), lambda i: (i, 0, 0))],
        out_specs=pl.BlockSpec((bb, c, w, h), lambda i: (i, 0, 0, 0)),
        out_shape=jax.ShapeDtypeStruct((b, c, w, h), f32),
        scratch_shapes=[pltpu.VMEM((b, c), f32), pltpu.VMEM((1, w, h), f32)],
    )(winners, winners, i1, i2)
    return out


# speedup vs baseline: 25.1152x; 1.1677x over previous
import jax
import jax.numpy as jnp
from jax.experimental import pallas as pl
from jax.experimental.pallas import tpu as pltpu

_LIFETIME_K = 5
_NEG = -3.0e38


def _make_iotas(w, h):
    fi_i = (jax.lax.broadcasted_iota(jnp.int32, (1, w, h), 1) * h
            + jax.lax.broadcasted_iota(jnp.int32, (1, w, h), 2))
    fi = fi_i.astype(jnp.float32)
    return fi, (w * h - 1.0) - fi


def _stage_a(x_ref, win_ref, i1_ref, i2_ref, fi_ref, rfi_ref):
    bb, c, w, h = x_ref.shape

    @pl.when(pl.program_id(0) == 0)
    def _():
        fi, rfi = _make_iotas(w, h)
        fi_ref[...] = fi
        rfi_ref[...] = rfi

    fi = fi_ref[...]
    rfi = rfi_ref[...]
    for j in range(bb):
        y = x_ref[j]
        m = jnp.max(jnp.max(y, axis=1), axis=-1)
        eqf = (y == m[:, None, None]).astype(jnp.float32)
        i2 = jnp.max(jnp.max(eqf * fi, axis=1), axis=-1)
        i1 = (w * h - 1.0) - jnp.max(jnp.max(eqf * rfi, axis=1), axis=-1)
        win_ref[j, 0, :] = m
        i1_ref[j, 0, :] = i1
        i2_ref[j, 0, :] = i2


def _stage_b(win_full_ref, win_ref, i1_ref, i2_ref, out_ref, m2_ref, fi_ref):
    bb = out_ref.shape[0]
    step = pl.program_id(0)

    @pl.when(step == 0)
    def _():
        w = win_full_ref[:, 0, :]
        ge = (w[:, None, :] >= w[None, :, :]).astype(jnp.float32)
        cnt = jnp.sum(ge, axis=0)
        thr = jnp.max(jnp.where(cnt >= _LIFETIME_K, w, _NEG), axis=0)
        m2_ref[...] = (w >= thr[None, :]).astype(jnp.float32)
        fi, _ = _make_iotas(out_ref.shape[2], out_ref.shape[3])
        fi_ref[...] = fi

    fi = fi_ref[...]
    for j in range(bb):
        b = step * bb + j
        wrow = win_ref[j, 0, :]
        i1 = i1_ref[j, 0, :]
        i2 = i2_ref[j, 0, :]
        m2row = m2_ref[pl.ds(b, 1), :][0]
        keep = (fi == i1[:, None, None]) | (fi == i2[:, None, None])
        val = (wrow * m2row)[:, None, None]
        out_ref[j] = jnp.where(keep, val, 0.0)


def kernel(x):
    b, c, w, h = x.shape
    f32 = jnp.float32
    bb = 2
    grid = b // bb
    winners, i1, i2 = pl.pallas_call(
        _stage_a,
        grid=(grid,),
        in_specs=[pl.BlockSpec((bb, c, w, h), lambda i: (i, 0, 0, 0))],
        out_specs=[pl.BlockSpec((bb, 1, c), lambda i: (i, 0, 0))] * 3,
        out_shape=[jax.ShapeDtypeStruct((b, 1, c), f32)] * 3,
        scratch_shapes=[pltpu.VMEM((1, w, h), f32)] * 2,
    )(x)
    out = pl.pallas_call(
        _stage_b,
        grid=(grid,),
        in_specs=[pl.BlockSpec((b, 1, c), lambda i: (0, 0, 0)),
                  pl.BlockSpec((bb, 1, c), lambda i: (i, 0, 0)),
                  pl.BlockSpec((bb, 1, c), lambda i: (i, 0, 0)),
                  pl.BlockSpec((bb, 1, c), lambda i: (i, 0, 0))],
        out_specs=pl.BlockSpec((bb, c, w, h), lambda i: (i, 0, 0, 0)),
        out_shape=jax.ShapeDtypeStruct((b, c, w, h), f32),
        scratch_shapes=[pltpu.VMEM((b, c), f32), pltpu.VMEM((1, w, h), f32)],
    )(winners, winners, i1, i2)
    return out


# speedup vs baseline: 133.4480x; 5.3134x over previous
import jax
import jax.numpy as jnp
from jax.experimental import pallas as pl
from jax.experimental.pallas import tpu as pltpu

_LIFETIME_K = 5
_NEG = -3.0e38


def _make_iotas(w, h):
    fi_i = (jax.lax.broadcasted_iota(jnp.int32, (w, h, 1), 0) * h
            + jax.lax.broadcasted_iota(jnp.int32, (w, h, 1), 1))
    fi = fi_i.astype(jnp.float32)
    return fi, (w * h - 1.0) - fi


def _stage_a(x_ref, win_ref, i1_ref, i2_ref, fi_ref, rfi_ref):
    bb, w, h, c = x_ref.shape

    @pl.when(pl.program_id(0) == 0)
    def _():
        fi, rfi = _make_iotas(w, h)
        fi_ref[...] = fi
        rfi_ref[...] = rfi

    fi = fi_ref[...]
    rfi = rfi_ref[...]
    for j in range(bb):
        y = x_ref[j]
        m = jnp.max(jnp.max(y, axis=0), axis=0)
        eqf = (y == m[None, None, :]).astype(jnp.float32)
        i2 = jnp.max(jnp.max(eqf * fi, axis=0), axis=0)
        i1 = (w * h - 1.0) - jnp.max(jnp.max(eqf * rfi, axis=0), axis=0)
        win_ref[j, 0, :] = m
        i1_ref[j, 0, :] = i1
        i2_ref[j, 0, :] = i2


def _stage_b(win_full_ref, win_ref, i1_ref, i2_ref, out_ref, m2_ref, fi_ref):
    bb, w, h, c = out_ref.shape
    step = pl.program_id(0)

    @pl.when(step == 0)
    def _():
        wv = win_full_ref[:, 0, :]
        ge = (wv[:, None, :] >= wv[None, :, :]).astype(jnp.float32)
        cnt = jnp.sum(ge, axis=0)
        thr = jnp.max(jnp.where(cnt >= _LIFETIME_K, wv, _NEG), axis=0)
        m2_ref[...] = (wv >= thr[None, :]).astype(jnp.float32)
        fi, _ = _make_iotas(w, h)
        fi_ref[...] = fi

    fi = fi_ref[...]
    for j in range(bb):
        b = step * bb + j
        wrow = win_ref[j, 0, :]
        i1 = i1_ref[j, 0, :]
        i2 = i2_ref[j, 0, :]
        m2row = m2_ref[pl.ds(b, 1), :][0]
        keep = (fi == i1[None, None, :]) | (fi == i2[None, None, :])
        val = (wrow * m2row)[None, None, :]
        out_ref[j] = jnp.where(keep, val, 0.0)


def kernel(x):
    b, c, w, h = x.shape
    f32 = jnp.float32
    bb = 4
    grid = b // bb
    xt = jnp.transpose(x, (0, 2, 3, 1))
    winners, i1, i2 = pl.pallas_call(
        _stage_a,
        grid=(grid,),
        in_specs=[pl.BlockSpec((bb, w, h, c), lambda i: (i, 0, 0, 0))],
        out_specs=[pl.BlockSpec((bb, 1, c), lambda i: (i, 0, 0))] * 3,
        out_shape=[jax.ShapeDtypeStruct((b, 1, c), f32)] * 3,
        scratch_shapes=[pltpu.VMEM((w, h, 1), f32)] * 2,
    )(xt)
    out_t = pl.pallas_call(
        _stage_b,
        grid=(grid,),
        in_specs=[pl.BlockSpec((b, 1, c), lambda i: (0, 0, 0)),
                  pl.BlockSpec((bb, 1, c), lambda i: (i, 0, 0)),
                  pl.BlockSpec((bb, 1, c), lambda i: (i, 0, 0)),
                  pl.BlockSpec((bb, 1, c), lambda i: (i, 0, 0))],
        out_specs=pl.BlockSpec((bb, w, h, c), lambda i: (i, 0, 0, 0)),
        out_shape=jax.ShapeDtypeStruct((b, w, h, c), f32),
        scratch_shapes=[pltpu.VMEM((b, c), f32), pltpu.VMEM((w, h, 1), f32)],
    )(winners, winners, i1, i2)
    return jnp.transpose(out_t, (0, 3, 1, 2))
